# Initial kernel scaffold; baseline (speedup 1.0000x reference)
import functools

import jax
import jax.numpy as jnp
from jax import lax
from jax.experimental import pallas as pl
from jax.experimental.pallas import tpu as pltpu
from jax.experimental.pallas import tpu_sc as plsc

N = 10000
E = 320000
D_IN = 128
HID = 128
NCLS = 40

NC = 2
NS = 16
NW = NC * NS
CHUNK = 128
CPW = 79
E_PAD = NW * CPW * CHUNK
N_PAD = 10016
RPS = N_PAD // NS
ORPS = N // NS
D2 = 64

_MESH = plsc.VectorSubcoreMesh(core_axis_name="c", subcore_axis_name="s")


def _sc_degree(dst_w, ones_hbm, z16):
    @functools.partial(
        pl.kernel,
        out_type=jax.ShapeDtypeStruct((NC, N, 16), jnp.float32),
        mesh=_MESH,
        scratch_types=[
            pltpu.VMEM((CPW, CHUNK), jnp.int32),
            pltpu.VMEM((CHUNK, 16), jnp.float32),
            pltpu.VMEM_SHARED((N_PAD, 16), jnp.float32),
            pltpu.SemaphoreType.DMA,
        ],
    )
    def k(dst_ref, ones_ref, z_ref, out_ref, dstv, onesv, shared, sem):
        c = lax.axis_index("c")
        s = lax.axis_index("s")
        w = s * NC + c
        pltpu.sync_copy(dst_ref.at[w], dstv)
        pltpu.sync_copy(ones_ref, onesv)
        pltpu.sync_copy(z_ref.at[pl.ds(s * RPS, RPS)],
                        shared.at[pl.ds(s * RPS, RPS)])
        plsc.subcore_barrier()

        @pl.loop(0, CPW)
        def _(j):
            pltpu.sync_copy(onesv, shared.at[dstv.at[j]], add=True)

        plsc.subcore_barrier()
        pltpu.sync_copy(shared.at[pl.ds(s * ORPS, ORPS)],
                        out_ref.at[c, pl.ds(s * ORPS, ORPS)])

    return k(dst_w, ones_hbm, z16)


def _sc_aggregate(g_hbm, src_w, dst_w, zeros_hbm, d):
    @functools.partial(
        pl.kernel,
        out_type=jax.ShapeDtypeStruct((NC, N, d), jnp.float32),
        mesh=_MESH,
        scratch_types=[
            pltpu.VMEM((CPW, CHUNK), jnp.int32),
            pltpu.VMEM((CPW, CHUNK), jnp.int32),
            pltpu.VMEM((CHUNK, d), jnp.float32),
            pltpu.VMEM_SHARED((N_PAD, d), jnp.float32),
            pltpu.SemaphoreType.DMA,
        ],
    )
    def k(g_ref, src_ref, dst_ref, z_ref, out_ref, srcv, dstv, rows, shared, sem):
        c = lax.axis_index("c")
        s = lax.axis_index("s")
        w = s * NC + c
        pltpu.sync_copy(src_ref.at[w], srcv)
        pltpu.sync_copy(dst_ref.at[w], dstv)
        pltpu.sync_copy(z_ref.at[pl.ds(s * RPS, RPS)],
                        shared.at[pl.ds(s * RPS, RPS)])
        plsc.subcore_barrier()

        @pl.loop(0, CPW)
        def _(j):
            pltpu.async_copy(g_ref.at[srcv.at[j]], rows, sem).wait()
            pltpu.sync_copy(rows, shared.at[dstv.at[j]], add=True)

        plsc.subcore_barrier()
        pltpu.sync_copy(shared.at[pl.ds(s * ORPS, ORPS)],
                        out_ref.at[c, pl.ds(s * ORPS, ORPS)])

    return k(g_hbm, src_w, dst_w, zeros_hbm)


def _tc_matmul(x, w):
    def body(x_ref, w_ref, o_ref):
        o_ref[...] = jnp.dot(x_ref[...], w_ref[...],
                             preferred_element_type=jnp.float32)

    m, k = x.shape
    n = w.shape[1]
    return pl.pallas_call(
        body,
        grid=(pl.cdiv(m, 128),),
        in_specs=[pl.BlockSpec((128, k), lambda i: (i, 0)),
                  pl.BlockSpec((k, n), lambda i: (0, 0))],
        out_specs=pl.BlockSpec((128, n), lambda i: (i, 0)),
        out_shape=jax.ShapeDtypeStruct((m, n), jnp.float32),
    )(x, w)


def _tc_scale(h, degs):
    def body(h_ref, d0_ref, d1_ref, g_ref, dinv_ref):
        deg = d0_ref[0, :, :1] + d1_ref[0, :, :1] + 1.0
        dinv = lax.rsqrt(deg)
        dinv_ref[...] = dinv
        g_ref[...] = h_ref[...] * dinv

    return pl.pallas_call(
        body,
        grid=(pl.cdiv(N, 128),),
        in_specs=[
            pl.BlockSpec((128, HID), lambda i: (i, 0)),
            pl.BlockSpec((1, 128, 16), lambda i: (0, i, 0)),
            pl.BlockSpec((1, 128, 16), lambda i: (1, i, 0)),
        ],
        out_specs=[
            pl.BlockSpec((128, HID), lambda i: (i, 0)),
            pl.BlockSpec((128, 1), lambda i: (i, 0)),
        ],
        out_shape=[
            jax.ShapeDtypeStruct((N, HID), jnp.float32),
            jax.ShapeDtypeStruct((N, 1), jnp.float32),
        ],
    )(h, degs)


def _tc_layer2_input(s1, g1, dinv, b1, w2p):
    def body(s0_ref, s1_ref, g_ref, dinv_ref, b_ref, w_ref, o_ref):
        agg = (s0_ref[0] + s1_ref[0] + g_ref[...]) * dinv_ref[...] + b_ref[...]
        act = jnp.maximum(agg, 0.0)
        o_ref[...] = jnp.dot(act, w_ref[...],
                             preferred_element_type=jnp.float32) * dinv_ref[...]

    return pl.pallas_call(
        body,
        grid=(pl.cdiv(N, 128),),
        in_specs=[
            pl.BlockSpec((1, 128, HID), lambda i: (0, i, 0)),
            pl.BlockSpec((1, 128, HID), lambda i: (1, i, 0)),
            pl.BlockSpec((128, HID), lambda i: (i, 0)),
            pl.BlockSpec((128, 1), lambda i: (i, 0)),
            pl.BlockSpec((HID,), lambda i: (0,)),
            pl.BlockSpec((HID, D2), lambda i: (0, 0)),
        ],
        out_specs=pl.BlockSpec((128, D2), lambda i: (i, 0)),
        out_shape=jax.ShapeDtypeStruct((N, D2), jnp.float32),
    )(s1, g1, dinv, b1, w2p)


def _tc_final(s2, g2, dinv, b2p):
    def body(s0_ref, s1_ref, g_ref, dinv_ref, b_ref, o_ref):
        o_ref[...] = ((s0_ref[0] + s1_ref[0] + g_ref[...]) * dinv_ref[...]
                      + b_ref[...])

    return pl.pallas_call(
        body,
        grid=(pl.cdiv(N, 128),),
        in_specs=[
            pl.BlockSpec((1, 128, D2), lambda i: (0, i, 0)),
            pl.BlockSpec((1, 128, D2), lambda i: (1, i, 0)),
            pl.BlockSpec((128, D2), lambda i: (i, 0)),
            pl.BlockSpec((128, 1), lambda i: (i, 0)),
            pl.BlockSpec((D2,), lambda i: (0,)),
        ],
        out_specs=pl.BlockSpec((128, D2), lambda i: (i, 0)),
        out_shape=jax.ShapeDtypeStruct((N, D2), jnp.float32),
    )(s2, g2, dinv, b2p)


def kernel(x, edge_index, W1, b1, W2, b2):
    src = edge_index[0]
    dst = edge_index[1]
    pad = E_PAD - E
    src_w = jnp.concatenate([src, jnp.zeros((pad,), jnp.int32)]).reshape(
        NW, CPW, CHUNK)
    dst_w = jnp.concatenate([dst, jnp.full((pad,), N, jnp.int32)]).reshape(
        NW, CPW, CHUNK)

    ones16 = jnp.ones((CHUNK, 16), jnp.float32)
    z16 = jnp.zeros((N_PAD, 16), jnp.float32)
    z128 = jnp.zeros((N_PAD, HID), jnp.float32)
    z64 = jnp.zeros((N_PAD, D2), jnp.float32)
    w2p = jnp.pad(W2, ((0, 0), (0, D2 - NCLS)))
    b2p = jnp.pad(b2, (0, D2 - NCLS))

    degs = _sc_degree(dst_w, ones16, z16)
    h1 = _tc_matmul(x, W1)
    g1, dinv = _tc_scale(h1, degs)
    s1 = _sc_aggregate(g1, src_w, dst_w, z128, HID)
    g2 = _tc_layer2_input(s1, g1, dinv, b1, w2p)
    s2 = _sc_aggregate(g2, src_w, dst_w, z64, D2)
    out = _tc_final(s2, g2, dinv, b2p)
    return out[:, :NCLS]


# speedup vs baseline: 10.7118x; 10.7118x over previous
import functools

import jax
import jax.numpy as jnp
from jax import lax
from jax.experimental import pallas as pl
from jax.experimental.pallas import tpu as pltpu
from jax.experimental.pallas import tpu_sc as plsc

N = 10000
E = 320000
D_IN = 128
HID = 128
NCLS = 40

NC = 2
NS = 16
NW = NC * NS
CHUNK = 128
CPW = 79
E_PAD = NW * CPW * CHUNK
N_PAD = 10112
RPS = N_PAD // NS
D2 = 128

_MESH = plsc.VectorSubcoreMesh(core_axis_name="c", subcore_axis_name="s")


def _sc_degree(dst_w, ones_hbm, z16):
    @functools.partial(
        pl.kernel,
        out_type=jax.ShapeDtypeStruct((NC, N_PAD, HID), jnp.float32),
        mesh=_MESH,
        scratch_types=[
            pltpu.VMEM((CPW, CHUNK), jnp.int32),
            pltpu.VMEM((CHUNK, HID), jnp.float32),
            pltpu.VMEM_SHARED((N_PAD, HID), jnp.float32),
            pltpu.SemaphoreType.DMA,
        ],
    )
    def k(dst_ref, ones_ref, z_ref, out_ref, dstv, onesv, shared, sem):
        c = lax.axis_index("c")
        s = lax.axis_index("s")
        w = s * NC + c
        pltpu.sync_copy(dst_ref.at[w], dstv)
        pltpu.sync_copy(ones_ref, onesv)
        pltpu.sync_copy(z_ref.at[pl.ds(s * RPS, RPS)],
                        shared.at[pl.ds(s * RPS, RPS)])
        plsc.subcore_barrier()

        @pl.loop(0, CPW)
        def _(j):
            pltpu.sync_copy(onesv, shared.at[dstv.at[j]], add=True)

        plsc.subcore_barrier()
        pltpu.sync_copy(shared.at[pl.ds(s * RPS, RPS)],
                        out_ref.at[c, pl.ds(s * RPS, RPS)])

    return k(dst_w, ones_hbm, z16)


def _sc_aggregate(g_hbm, src_w, dst_w, zeros_hbm, d):
    @functools.partial(
        pl.kernel,
        out_type=jax.ShapeDtypeStruct((NC, N_PAD, d), jnp.float32),
        mesh=_MESH,
        scratch_types=[
            pltpu.VMEM((CPW, CHUNK), jnp.int32),
            pltpu.VMEM((CPW, CHUNK), jnp.int32),
            pltpu.VMEM((CHUNK, d), jnp.float32),
            pltpu.VMEM_SHARED((N_PAD, d), jnp.float32),
            pltpu.SemaphoreType.DMA,
        ],
    )
    def k(g_ref, src_ref, dst_ref, z_ref, out_ref, srcv, dstv, rows, shared, sem):
        c = lax.axis_index("c")
        s = lax.axis_index("s")
        w = s * NC + c
        pltpu.sync_copy(src_ref.at[w], srcv)
        pltpu.sync_copy(dst_ref.at[w], dstv)
        pltpu.sync_copy(z_ref.at[pl.ds(s * RPS, RPS)],
                        shared.at[pl.ds(s * RPS, RPS)])
        plsc.subcore_barrier()

        @pl.loop(0, CPW)
        def _(j):
            pltpu.async_copy(g_ref.at[srcv.at[j]], rows, sem).wait()
            pltpu.sync_copy(rows, shared.at[dstv.at[j]], add=True)

        plsc.subcore_barrier()
        pltpu.sync_copy(shared.at[pl.ds(s * RPS, RPS)],
                        out_ref.at[c, pl.ds(s * RPS, RPS)])

    return k(g_hbm, src_w, dst_w, zeros_hbm)


def _tc_matmul(x, w):
    def body(x_ref, w_ref, o_ref):
        o_ref[...] = jnp.dot(x_ref[...], w_ref[...],
                             preferred_element_type=jnp.float32)

    m, k = x.shape
    n = w.shape[1]
    return pl.pallas_call(
        body,
        grid=(pl.cdiv(m, 128),),
        in_specs=[pl.BlockSpec((128, k), lambda i: (i, 0)),
                  pl.BlockSpec((k, n), lambda i: (0, 0))],
        out_specs=pl.BlockSpec((128, n), lambda i: (i, 0)),
        out_shape=jax.ShapeDtypeStruct((m, n), jnp.float32),
    )(x, w)


def _tc_scale(h, degs):
    def body(h_ref, d_ref, g_ref, dinv_ref):
        deg = d_ref[0, :, :1] + d_ref[1, :, :1] + 1.0
        dinv = lax.rsqrt(deg)
        dinv_ref[...] = dinv
        g_ref[...] = h_ref[...] * dinv

    return pl.pallas_call(
        body,
        grid=(pl.cdiv(N, 128),),
        in_specs=[
            pl.BlockSpec((128, HID), lambda i: (i, 0)),
            pl.BlockSpec((2, 128, HID), lambda i: (0, i, 0)),
        ],
        out_specs=[
            pl.BlockSpec((128, HID), lambda i: (i, 0)),
            pl.BlockSpec((128, 1), lambda i: (i, 0)),
        ],
        out_shape=[
            jax.ShapeDtypeStruct((N, HID), jnp.float32),
            jax.ShapeDtypeStruct((N, 1), jnp.float32),
        ],
    )(h, degs)


def _tc_layer2_input(s1, g1, dinv, b1, w2p):
    def body(s_ref, g_ref, dinv_ref, b_ref, w_ref, o_ref):
        agg = (s_ref[0] + s_ref[1] + g_ref[...]) * dinv_ref[...] + b_ref[...]
        act = jnp.maximum(agg, 0.0)
        o_ref[...] = jnp.dot(act, w_ref[...],
                             preferred_element_type=jnp.float32) * dinv_ref[...]

    return pl.pallas_call(
        body,
        grid=(pl.cdiv(N, 128),),
        in_specs=[
            pl.BlockSpec((2, 128, HID), lambda i: (0, i, 0)),
            pl.BlockSpec((128, HID), lambda i: (i, 0)),
            pl.BlockSpec((128, 1), lambda i: (i, 0)),
            pl.BlockSpec((HID,), lambda i: (0,)),
            pl.BlockSpec((HID, D2), lambda i: (0, 0)),
        ],
        out_specs=pl.BlockSpec((128, D2), lambda i: (i, 0)),
        out_shape=jax.ShapeDtypeStruct((N, D2), jnp.float32),
    )(s1, g1, dinv, b1, w2p)


def _tc_final(s2, g2, dinv, b2p):
    def body(s_ref, g_ref, dinv_ref, b_ref, o_ref):
        o_ref[...] = ((s_ref[0] + s_ref[1] + g_ref[...]) * dinv_ref[...]
                      + b_ref[...])

    return pl.pallas_call(
        body,
        grid=(pl.cdiv(N, 128),),
        in_specs=[
            pl.BlockSpec((2, 128, D2), lambda i: (0, i, 0)),
            pl.BlockSpec((128, D2), lambda i: (i, 0)),
            pl.BlockSpec((128, 1), lambda i: (i, 0)),
            pl.BlockSpec((D2,), lambda i: (0,)),
        ],
        out_specs=pl.BlockSpec((128, D2), lambda i: (i, 0)),
        out_shape=jax.ShapeDtypeStruct((N, D2), jnp.float32),
    )(s2, g2, dinv, b2p)


def kernel(x, edge_index, W1, b1, W2, b2):
    src = edge_index[0]
    dst = edge_index[1]
    pad = E_PAD - E
    src_w = jnp.concatenate([src, jnp.zeros((pad,), jnp.int32)]).reshape(
        NW, CPW, CHUNK)
    dst_w = jnp.concatenate([dst, jnp.full((pad,), N, jnp.int32)]).reshape(
        NW, CPW, CHUNK)

    ones = jnp.ones((CHUNK, HID), jnp.float32)
    z128 = jnp.zeros((N_PAD, HID), jnp.float32)
    z64 = jnp.zeros((N_PAD, D2), jnp.float32)
    w2p = jnp.pad(W2, ((0, 0), (0, D2 - NCLS)))
    b2p = jnp.pad(b2, (0, D2 - NCLS))

    degs = _sc_degree(dst_w, ones, z128)
    h1 = _tc_matmul(x, W1)
    g1, dinv = _tc_scale(h1, degs)
    s1 = _sc_aggregate(g1, src_w, dst_w, z128, HID)
    g2 = _tc_layer2_input(s1, g1, dinv, b1, w2p)
    s2 = _sc_aggregate(g2, src_w, dst_w, z64, D2)
    out = _tc_final(s2, g2, dinv, b2p)
    return out[:, :NCLS]
